# baseline (speedup 1.0000x reference)
import functools

import jax
import jax.numpy as jnp
from jax import lax
from jax.experimental import pallas as pl
from jax.experimental.pallas import tpu as pltpu
from jax.experimental.pallas import tpu_sc as plsc

D = 64
L = 16
NC = 2
NS = 16
NW = NC * NS
CW = 512
NCHUNK = 999936 // CW
TAIL = 999936
RING = 64
RP = 128


NQ = 62
NR = 2
UW = 512


@functools.cache
def _make_gather_kernel(B, V):
    mesh = plsc.VectorSubcoreMesh(core_axis_name="c", subcore_axis_name="s")
    NV = B // L

    @functools.partial(
        pl.kernel,
        mesh=mesh,
        out_type=(jax.ShapeDtypeStruct((B * RP,), jnp.float32),
                  jax.ShapeDtypeStruct((B * RP,), jnp.float32)),
        compiler_params=pltpu.CompilerParams(
            needs_layout_passes=False, use_tc_tiling_on_sc=True),
        scratch_types=[
            pltpu.VMEM((B,), jnp.int32),
            pltpu.VMEM((B + L,), jnp.int32),
            pltpu.VMEM((B + L,), jnp.int32),
            pltpu.VMEM((NR * D, UW), jnp.float32),
            pltpu.VMEM((RING * RP,), jnp.float32),
            pltpu.SMEM((64,), jnp.int32),
            pltpu.SMEM((64,), jnp.int32),
        ] + [pltpu.SemaphoreType.DMA] * (NR + 1),
    )
    def kern(uids_hbm, iids_hbm, utT_hbm, itT_hbm, ustage_hbm, istage_hbm,
             ids_v, lst, lst2, qbuf, ring_v, counts_s, pos_s, *sems):
        semq = sems[:NR]
        semr = sems[NR]
        wid = lax.axis_index("s") * NC + lax.axis_index("c")
        lanes = lax.iota(jnp.int32, L)

        def one_table(ids_hbm, tT_hbm, stage_hbm):
            pltpu.sync_copy(ids_hbm, ids_v)

            def scan_ids(j, off):
                idv = ids_v[pl.ds(j * L, L)]
                bv = jnp.full((L,), j * L, jnp.int32) + lanes
                keep = ((idv >> 9) & (NW - 1)) == wid
                cnt = plsc.all_reduce_population_count(keep)[0]
                packed = (bv << 15) | ((idv >> 14) << 9) | (idv & 0x1FF)
                plsc.store_compressed(lst.at[pl.ds(off, L)], packed, mask=keep)
                return off + cnt

            n = lax.fori_loop(0, NV, scan_ids, 0)
            nv = (n + L - 1) // L

            def zero(i, c):
                counts_s[i] = 0
                return c

            lax.fori_loop(0, 64, zero, 0)

            def hist(j, c):
                vv = lst[pl.ds(j * L, L)]
                for lane in range(L):
                    kk = (vv[lane] >> 9) & 0x3F
                    inc = jnp.where(j * L + lane < n, 1, 0)
                    counts_s[kk] = counts_s[kk] + inc
                return c

            lax.fori_loop(0, nv, hist, 0)

            def pref(k, run):
                pos_s[k] = run
                return run + counts_s[k]

            lax.fori_loop(0, 64, pref, 0)

            def scat(j, c):
                vv = lst[pl.ds(j * L, L)]
                for lane in range(L):
                    kk = (vv[lane] >> 9) & 0x3F
                    p = pos_s[kk]
                    ok = j * L + lane < n
                    onehot = (lanes == lane) & ok
                    plsc.store_scatter(
                        lst2, [jnp.full((L,), p, jnp.int32)], vv, mask=onehot)
                    pos_s[kk] = p + jnp.where(ok, 1, 0)
                return c

            lax.fori_loop(0, nv, scat, 0)

            def issue(Q, sem, boff):
                g = wid + Q * NW
                col = g * CW

                @pl.when(g < NCHUNK)
                def _():
                    for i in range(D // 8):
                        pltpu.async_copy(
                            tT_hbm.at[pl.ds(i * 8, 8), pl.ds(col, UW)],
                            qbuf.at[pl.ds(boff + i * 8, 8), :], sem)

            def wait_q(Q, sem, boff):
                g = wid + Q * NW
                col = g * CW

                @pl.when(g < NCHUNK)
                def _():
                    for i in range(D // 8):
                        pltpu.make_async_copy(
                            tT_hbm.at[pl.ds(i * 8, 8), pl.ds(col, UW)],
                            qbuf.at[pl.ds(boff + i * 8, 8), :], sem).wait()

            def process(Q, boff, pending):
                g = wid + Q * NW
                cnt = counts_s[Q]
                p0 = pos_s[Q] - cnt

                def serve(pending):
                    def scan_seg(j, pend):
                        @pl.when(pend > RING - L)
                        def _():
                            def drain1(_i, c):
                                pltpu.make_async_copy(
                                    stage_hbm.at[pl.ds(0, RP)],
                                    ring_v.at[pl.ds(0, RP)],
                                    semr).wait()
                                return c

                            lax.fori_loop(0, pend, drain1, 0)

                        pend = jnp.where(pend > RING - L, 0, pend)
                        vv = lst2[pl.ds(p0 + j * L, L)]
                        act = (lanes + j * L) < cnt
                        act32 = act.astype(jnp.int32)
                        for lane in range(L):
                            mbit = act32[lane]
                            slot = pend

                            @pl.when(mbit == 1)
                            def _():
                                q = vv[lane] & 511
                                b = vv[lane] >> 15
                                qs = jnp.full((L,), q, jnp.int32)
                                dbase = jnp.full((L,), boff, jnp.int32)
                                for jj in range(D // L):
                                    dv = dbase + lanes + jj * L
                                    vals = plsc.load_gather(qbuf, [dv, qs])
                                    ring_v[pl.ds(slot * RP + jj * L, L)] = vals
                                pltpu.async_copy(
                                    ring_v.at[pl.ds(slot * RP, RP)],
                                    stage_hbm.at[pl.ds(b * RP, RP)],
                                    semr)

                            pend = pend + mbit
                        return pend

                    return lax.fori_loop(0, (cnt + L - 1) // L, scan_seg,
                                         pending)

                return lax.cond((g < NCHUNK) & (cnt > 0), serve,
                                lambda p: p, pending)

            for r in range(NR - 1):
                issue(r, semq[r], r * D)

            def group(q8, pending):
                qb = q8 * NR
                for r in range(NR):
                    Q = qb + r
                    wait_q(Q, semq[r], r * D)
                    rn = (r + NR - 1) % NR
                    issue(Q + NR - 1, semq[rn], rn * D)
                    pending = process(Q, r * D, pending)
                return pending

            pending = lax.fori_loop(0, NQ // NR, group, 0)

            def final_drain(_i, carry):
                pltpu.make_async_copy(
                    stage_hbm.at[pl.ds(0, RP)], ring_v.at[pl.ds(0, RP)],
                    semr).wait()
                return carry

            lax.fori_loop(0, pending, final_drain, 0)

        one_table(uids_hbm, utT_hbm, ustage_hbm)
        one_table(iids_hbm, itT_hbm, istage_hbm)

    return kern


@functools.cache
def _make_dot_kernel(B):
    mesh = plsc.VectorSubcoreMesh(core_axis_name="c", subcore_axis_name="s")
    BPW = B // NW
    GRP = 256

    @functools.partial(
        pl.kernel,
        mesh=mesh,
        out_type=jax.ShapeDtypeStruct((B,), jnp.float32),
        compiler_params=pltpu.CompilerParams(
            needs_layout_passes=False, use_tc_tiling_on_sc=False),
        scratch_types=[
            pltpu.VMEM((GRP * RP,), jnp.float32),
            pltpu.VMEM((GRP * RP,), jnp.float32),
            pltpu.VMEM((BPW,), jnp.int32),
            pltpu.VMEM((BPW,), jnp.int32),
            pltpu.VMEM((D * D,), jnp.float32),
            pltpu.VMEM((D * D,), jnp.float32),
            pltpu.VMEM((BPW,), jnp.float32),
        ],
    )
    def kern(uids_hbm, iids_hbm, ustage_hbm, istage_hbm, utail_hbm, itail_hbm,
             out_hbm, uv, iv, uidv, iidv, utl, itl, out_v):
        wid = lax.axis_index("s") * NC + lax.axis_index("c")
        base = wid * BPW
        lanes = lax.iota(jnp.int32, L)

        pltpu.sync_copy(uids_hbm.at[pl.ds(base, BPW)], uidv)
        pltpu.sync_copy(iids_hbm.at[pl.ds(base, BPW)], iidv)
        pltpu.sync_copy(utail_hbm, utl)
        pltpu.sync_copy(itail_hbm, itl)

        for grp in range(BPW // GRP):
            g0 = base + grp * GRP
            pltpu.sync_copy(ustage_hbm.at[pl.ds(g0 * RP, GRP * RP)], uv)
            pltpu.sync_copy(istage_hbm.at[pl.ds(g0 * RP, GRP * RP)], iv)

            def body(it, carry):
                r0 = it * L
                res = jnp.zeros((L,), jnp.float32)
                for kk in range(L):
                    r = r0 + kk
                    acc = jnp.zeros((L,), jnp.float32)
                    for j in range(D // L):
                        pu = uv[pl.ds(r * RP + j * L, L)]
                        pi = iv[pl.ds(r * RP + j * L, L)]
                        acc = acc + pu * pi
                    res = jnp.where(lanes == kk, jnp.sum(acc), res)

                ob = grp * GRP + r0
                uidvec = uidv[pl.ds(ob, L)]
                iidvec = iidv[pl.ds(ob, L)]
                tl = (uidvec >= TAIL) | (iidvec >= TAIL)
                tl32 = tl.astype(jnp.int32)
                nt = plsc.all_reduce_population_count(tl)[0]

                @pl.when(nt > 0)
                def _():
                    fixed = res
                    for lane in range(L):
                        @pl.when(tl32[lane] == 1)
                        def _():
                            r = r0 + lane
                            uid = uidvec[lane]
                            iid = iidvec[lane]
                            uo = lax.max(uid - TAIL, 0) * D
                            io = lax.max(iid - TAIL, 0) * D
                            acc2 = jnp.zeros((L,), jnp.float32)
                            for j in range(D // L):
                                us = uv[pl.ds(r * RP + j * L, L)]
                                ut = utl[pl.ds(uo + j * L, L)]
                                uu = jnp.where(uid >= TAIL, ut, us)
                                ss = iv[pl.ds(r * RP + j * L, L)]
                                st = itl[pl.ds(io + j * L, L)]
                                ii = jnp.where(iid >= TAIL, st, ss)
                                acc2 = acc2 + uu * ii
                            s = jnp.sum(acc2)
                            cur = out_v[pl.ds(ob, L)]
                            out_v[pl.ds(ob, L)] = jnp.where(
                                lanes == lane, s, cur)

                    out_v[pl.ds(ob, L)] = jnp.where(
                        tl, out_v[pl.ds(ob, L)], fixed)

                @pl.when(nt == 0)
                def _():
                    out_v[pl.ds(ob, L)] = res

                return carry

            lax.fori_loop(0, GRP // L, body, 0)

        pltpu.sync_copy(out_v, out_hbm.at[pl.ds(base, BPW)])

    return kern


def kernel(user_ids, item_ids, user_table, item_table):
    B = user_ids.shape[0]
    V = user_table.shape[0]
    uids = user_ids.astype(jnp.int32)
    iids = item_ids.astype(jnp.int32)
    utail = user_table[TAIL:, :].reshape(-1)
    itail = item_table[TAIL:, :].reshape(-1)
    gather = _make_gather_kernel(B, V)
    ustage, istage = gather(uids, iids, user_table.T, item_table.T)
    dot = _make_dot_kernel(B)
    return dot(uids, iids, ustage, istage, utail, itail)


# speedup vs baseline: 1.0107x; 1.0107x over previous
import functools

import jax
import jax.numpy as jnp
from jax import lax
from jax.experimental import pallas as pl
from jax.experimental.pallas import tpu as pltpu
from jax.experimental.pallas import tpu_sc as plsc

D = 64
L = 16
NC = 2
NS = 16
NW = NC * NS
CW = 512
NCHUNK = 999936 // CW
TAIL = 999936
RING = 64
RP = 128


NQ = 62
NR = 2
UW = 512


@functools.cache
def _make_gather_kernel(B, V):
    mesh = plsc.VectorSubcoreMesh(core_axis_name="c", subcore_axis_name="s")
    NV = B // L

    @functools.partial(
        pl.kernel,
        mesh=mesh,
        out_type=(jax.ShapeDtypeStruct((B * D,), jnp.float32),
                  jax.ShapeDtypeStruct((B * D,), jnp.float32)),
        compiler_params=pltpu.CompilerParams(
            needs_layout_passes=False, use_tc_tiling_on_sc=True),
        scratch_types=[
            pltpu.VMEM((B,), jnp.int32),
            pltpu.VMEM((B + L,), jnp.int32),
            pltpu.VMEM((B + L,), jnp.int32),
            pltpu.VMEM((NR * D, UW), jnp.float32),
            pltpu.VMEM((RING * RP,), jnp.float32),
            pltpu.SMEM((64,), jnp.int32),
            pltpu.SMEM((64,), jnp.int32),
        ] + [pltpu.SemaphoreType.DMA] * (NR + 1),
    )
    def kern(uids_hbm, iids_hbm, utT_hbm, itT_hbm, ustage_hbm, istage_hbm,
             ids_v, lst, lst2, qbuf, ring_v, counts_s, pos_s, *sems):
        semq = sems[:NR]
        semr = sems[NR]
        wid = lax.axis_index("s") * NC + lax.axis_index("c")
        lanes = lax.iota(jnp.int32, L)

        def one_table(ids_hbm, tT_hbm, stage_hbm):
            pltpu.sync_copy(ids_hbm, ids_v)

            def scan_ids(j, off):
                idv = ids_v[pl.ds(j * L, L)]
                bv = jnp.full((L,), j * L, jnp.int32) + lanes
                keep = ((idv >> 9) & (NW - 1)) == wid
                cnt = plsc.all_reduce_population_count(keep)[0]
                packed = (bv << 15) | ((idv >> 14) << 9) | (idv & 0x1FF)
                plsc.store_compressed(lst.at[pl.ds(off, L)], packed, mask=keep)
                return off + cnt

            n = lax.fori_loop(0, NV, scan_ids, 0)
            nv = (n + L - 1) // L

            def zero(i, c):
                counts_s[i] = 0
                return c

            lax.fori_loop(0, 64, zero, 0)

            def hist(j, c):
                vv = lst[pl.ds(j * L, L)]
                for lane in range(L):
                    kk = (vv[lane] >> 9) & 0x3F
                    inc = jnp.where(j * L + lane < n, 1, 0)
                    counts_s[kk] = counts_s[kk] + inc
                return c

            lax.fori_loop(0, nv, hist, 0)

            def pref(k, run):
                pos_s[k] = run
                return run + counts_s[k]

            lax.fori_loop(0, 64, pref, 0)

            def scat(j, c):
                vv = lst[pl.ds(j * L, L)]
                for lane in range(L):
                    kk = (vv[lane] >> 9) & 0x3F
                    p = pos_s[kk]
                    ok = j * L + lane < n
                    onehot = (lanes == lane) & ok
                    plsc.store_scatter(
                        lst2, [jnp.full((L,), p, jnp.int32)], vv, mask=onehot)
                    pos_s[kk] = p + jnp.where(ok, 1, 0)
                return c

            lax.fori_loop(0, nv, scat, 0)

            def issue(Q, sem, boff):
                g = wid + Q * NW
                col = g * CW

                @pl.when(g < NCHUNK)
                def _():
                    for i in range(D // 8):
                        pltpu.async_copy(
                            tT_hbm.at[pl.ds(i * 8, 8), pl.ds(col, UW)],
                            qbuf.at[pl.ds(boff + i * 8, 8), :], sem)

            def wait_q(Q, sem, boff):
                g = wid + Q * NW
                col = g * CW

                @pl.when(g < NCHUNK)
                def _():
                    for i in range(D // 8):
                        pltpu.make_async_copy(
                            tT_hbm.at[pl.ds(i * 8, 8), pl.ds(col, UW)],
                            qbuf.at[pl.ds(boff + i * 8, 8), :], sem).wait()

            def process(Q, boff, pending):
                g = wid + Q * NW
                cnt = counts_s[Q]
                p0 = pos_s[Q] - cnt

                def serve(pending):
                    def scan_seg(j, pend):
                        @pl.when(pend > RING - L)
                        def _():
                            def drain1(_i, c):
                                pltpu.make_async_copy(
                                    stage_hbm.at[pl.ds(0, D)],
                                    ring_v.at[pl.ds(0, D)],
                                    semr).wait()
                                return c

                            lax.fori_loop(0, pend, drain1, 0)

                        pend = jnp.where(pend > RING - L, 0, pend)
                        vv = lst2[pl.ds(p0 + j * L, L)]
                        act = (lanes + j * L) < cnt
                        act32 = act.astype(jnp.int32)
                        for lane in range(L):
                            mbit = act32[lane]
                            slot = pend

                            @pl.when(mbit == 1)
                            def _():
                                q = vv[lane] & 511
                                b = vv[lane] >> 15
                                qs = jnp.full((L,), q, jnp.int32)
                                dbase = jnp.full((L,), boff, jnp.int32)
                                for jj in range(D // L):
                                    dv = dbase + lanes + jj * L
                                    vals = plsc.load_gather(qbuf, [dv, qs])
                                    ring_v[pl.ds(slot * RP + jj * L, L)] = vals
                                pltpu.async_copy(
                                    ring_v.at[pl.ds(slot * RP, D)],
                                    stage_hbm.at[pl.ds(b * D, D)],
                                    semr)

                            pend = pend + mbit
                        return pend

                    return lax.fori_loop(0, (cnt + L - 1) // L, scan_seg,
                                         pending)

                return lax.cond((g < NCHUNK) & (cnt > 0), serve,
                                lambda p: p, pending)

            for r in range(NR - 1):
                issue(r, semq[r], r * D)

            def group(q8, pending):
                qb = q8 * NR
                for r in range(NR):
                    Q = qb + r
                    wait_q(Q, semq[r], r * D)
                    rn = (r + NR - 1) % NR
                    issue(Q + NR - 1, semq[rn], rn * D)
                    pending = process(Q, r * D, pending)
                return pending

            pending = lax.fori_loop(0, NQ // NR, group, 0)

            def final_drain(_i, carry):
                pltpu.make_async_copy(
                    stage_hbm.at[pl.ds(0, D)], ring_v.at[pl.ds(0, D)],
                    semr).wait()
                return carry

            lax.fori_loop(0, pending, final_drain, 0)

        one_table(uids_hbm, utT_hbm, ustage_hbm)
        one_table(iids_hbm, itT_hbm, istage_hbm)

    return kern


@functools.cache
def _make_dot_kernel(B):
    mesh = plsc.VectorSubcoreMesh(core_axis_name="c", subcore_axis_name="s")
    BPW = B // NW
    GRP = 256

    @functools.partial(
        pl.kernel,
        mesh=mesh,
        out_type=jax.ShapeDtypeStruct((B,), jnp.float32),
        compiler_params=pltpu.CompilerParams(
            needs_layout_passes=False, use_tc_tiling_on_sc=False),
        scratch_types=[
            pltpu.VMEM((GRP * D,), jnp.float32),
            pltpu.VMEM((GRP * D,), jnp.float32),
            pltpu.VMEM((BPW,), jnp.int32),
            pltpu.VMEM((BPW,), jnp.int32),
            pltpu.VMEM((D * D,), jnp.float32),
            pltpu.VMEM((D * D,), jnp.float32),
            pltpu.VMEM((BPW,), jnp.float32),
        ],
    )
    def kern(uids_hbm, iids_hbm, ustage_hbm, istage_hbm, utail_hbm, itail_hbm,
             out_hbm, uv, iv, uidv, iidv, utl, itl, out_v):
        wid = lax.axis_index("s") * NC + lax.axis_index("c")
        base = wid * BPW
        lanes = lax.iota(jnp.int32, L)

        pltpu.sync_copy(uids_hbm.at[pl.ds(base, BPW)], uidv)
        pltpu.sync_copy(iids_hbm.at[pl.ds(base, BPW)], iidv)
        pltpu.sync_copy(utail_hbm, utl)
        pltpu.sync_copy(itail_hbm, itl)

        for grp in range(BPW // GRP):
            g0 = base + grp * GRP
            pltpu.sync_copy(ustage_hbm.at[pl.ds(g0 * D, GRP * D)], uv)
            pltpu.sync_copy(istage_hbm.at[pl.ds(g0 * D, GRP * D)], iv)

            def body(it, carry):
                r0 = it * L
                res = jnp.zeros((L,), jnp.float32)
                for kk in range(L):
                    r = r0 + kk
                    acc = jnp.zeros((L,), jnp.float32)
                    for j in range(D // L):
                        pu = uv[pl.ds(r * D + j * L, L)]
                        pi = iv[pl.ds(r * D + j * L, L)]
                        acc = acc + pu * pi
                    res = jnp.where(lanes == kk, jnp.sum(acc), res)

                ob = grp * GRP + r0
                uidvec = uidv[pl.ds(ob, L)]
                iidvec = iidv[pl.ds(ob, L)]
                tl = (uidvec >= TAIL) | (iidvec >= TAIL)
                tl32 = tl.astype(jnp.int32)
                nt = plsc.all_reduce_population_count(tl)[0]

                @pl.when(nt > 0)
                def _():
                    fixed = res
                    for lane in range(L):
                        @pl.when(tl32[lane] == 1)
                        def _():
                            r = r0 + lane
                            uid = uidvec[lane]
                            iid = iidvec[lane]
                            uo = lax.max(uid - TAIL, 0) * D
                            io = lax.max(iid - TAIL, 0) * D
                            acc2 = jnp.zeros((L,), jnp.float32)
                            for j in range(D // L):
                                us = uv[pl.ds(r * D + j * L, L)]
                                ut = utl[pl.ds(uo + j * L, L)]
                                uu = jnp.where(uid >= TAIL, ut, us)
                                ss = iv[pl.ds(r * D + j * L, L)]
                                st = itl[pl.ds(io + j * L, L)]
                                ii = jnp.where(iid >= TAIL, st, ss)
                                acc2 = acc2 + uu * ii
                            s = jnp.sum(acc2)
                            cur = out_v[pl.ds(ob, L)]
                            out_v[pl.ds(ob, L)] = jnp.where(
                                lanes == lane, s, cur)

                    out_v[pl.ds(ob, L)] = jnp.where(
                        tl, out_v[pl.ds(ob, L)], fixed)

                @pl.when(nt == 0)
                def _():
                    out_v[pl.ds(ob, L)] = res

                return carry

            lax.fori_loop(0, GRP // L, body, 0)

        pltpu.sync_copy(out_v, out_hbm.at[pl.ds(base, BPW)])

    return kern


def kernel(user_ids, item_ids, user_table, item_table):
    B = user_ids.shape[0]
    V = user_table.shape[0]
    uids = user_ids.astype(jnp.int32)
    iids = item_ids.astype(jnp.int32)
    utail = user_table[TAIL:, :].reshape(-1)
    itail = item_table[TAIL:, :].reshape(-1)
    gather = _make_gather_kernel(B, V)
    ustage, istage = gather(uids, iids, user_table.T, item_table.T)
    dot = _make_dot_kernel(B)
    return dot(uids, iids, ustage, istage, utail, itail)


# speedup vs baseline: 1.0180x; 1.0073x over previous
import functools

import jax
import jax.numpy as jnp
from jax import lax
from jax.experimental import pallas as pl
from jax.experimental.pallas import tpu as pltpu
from jax.experimental.pallas import tpu_sc as plsc

D = 64
L = 16
NC = 2
NS = 16
NW = NC * NS
CW = 512
NCHUNK = 999936 // CW
TAIL = 999936
RING = 64
RP = 128


NQ = 62
NR = 2
UW = 512


@functools.cache
def _make_gather_kernel(B, V):
    mesh = plsc.VectorSubcoreMesh(core_axis_name="c", subcore_axis_name="s")
    NV = B // L

    @functools.partial(
        pl.kernel,
        mesh=mesh,
        out_type=(jax.ShapeDtypeStruct((B * D,), jnp.float32),
                  jax.ShapeDtypeStruct((B * D,), jnp.float32)),
        compiler_params=pltpu.CompilerParams(
            needs_layout_passes=False, use_tc_tiling_on_sc=True),
        scratch_types=[
            pltpu.VMEM((B,), jnp.int32),
            pltpu.VMEM((B + L,), jnp.int32),
            pltpu.VMEM((B + L,), jnp.int32),
            pltpu.VMEM((NR * D, UW), jnp.float32),
            pltpu.VMEM((RING * RP,), jnp.float32),
            pltpu.SMEM((64,), jnp.int32),
            pltpu.SMEM((64,), jnp.int32),
        ] + [pltpu.SemaphoreType.DMA] * (NR + 1),
    )
    def kern(uids_hbm, iids_hbm, utT_hbm, itT_hbm, ustage_hbm, istage_hbm,
             ids_v, lst, lst2, qbuf, ring_v, counts_s, pos_s, *sems):
        semq = sems[:NR]
        semr = sems[NR]
        wid = lax.axis_index("s") * NC + lax.axis_index("c")
        lanes = lax.iota(jnp.int32, L)

        def one_table(ids_hbm, tT_hbm, stage_hbm, prime, finish):
            prime()
            pltpu.sync_copy(ids_hbm, ids_v)

            def scan_ids(j, off):
                idv = ids_v[pl.ds(j * L, L)]
                bv = jnp.full((L,), j * L, jnp.int32) + lanes
                keep = ((idv >> 9) & (NW - 1)) == wid
                cnt = plsc.all_reduce_population_count(keep)[0]
                packed = (bv << 15) | ((idv >> 14) << 9) | (idv & 0x1FF)
                plsc.store_compressed(lst.at[pl.ds(off, L)], packed, mask=keep)
                return off + cnt

            n = lax.fori_loop(0, NV, scan_ids, 0)
            nv = (n + L - 1) // L

            def zero(i, c):
                counts_s[i] = 0
                return c

            lax.fori_loop(0, 64, zero, 0)

            def hist(j, c):
                vv = lst[pl.ds(j * L, L)]
                for lane in range(L):
                    kk = (vv[lane] >> 9) & 0x3F
                    inc = jnp.where(j * L + lane < n, 1, 0)
                    counts_s[kk] = counts_s[kk] + inc
                return c

            lax.fori_loop(0, nv, hist, 0)

            def pref(k, run):
                pos_s[k] = run
                return run + counts_s[k]

            lax.fori_loop(0, 64, pref, 0)

            def scat(j, c):
                vv = lst[pl.ds(j * L, L)]
                for lane in range(L):
                    kk = (vv[lane] >> 9) & 0x3F
                    p = pos_s[kk]
                    ok = j * L + lane < n
                    onehot = (lanes == lane) & ok
                    plsc.store_scatter(
                        lst2, [jnp.full((L,), p, jnp.int32)], vv, mask=onehot)
                    pos_s[kk] = p + jnp.where(ok, 1, 0)
                return c

            lax.fori_loop(0, nv, scat, 0)

            def issue(Q, sem, boff):
                g = wid + Q * NW
                col = g * CW

                @pl.when(g < NCHUNK)
                def _():
                    for i in range(D // 8):
                        pltpu.async_copy(
                            tT_hbm.at[pl.ds(i * 8, 8), pl.ds(col, UW)],
                            qbuf.at[pl.ds(boff + i * 8, 8), :], sem)

            def wait_q(Q, sem, boff):
                g = wid + Q * NW
                col = g * CW

                @pl.when(g < NCHUNK)
                def _():
                    for i in range(D // 8):
                        pltpu.make_async_copy(
                            tT_hbm.at[pl.ds(i * 8, 8), pl.ds(col, UW)],
                            qbuf.at[pl.ds(boff + i * 8, 8), :], sem).wait()

            def process(Q, boff, pending):
                g = wid + Q * NW
                cnt = counts_s[Q]
                p0 = pos_s[Q] - cnt

                def serve(pending):
                    def scan_seg(j, pend):
                        @pl.when(pend > RING - L)
                        def _():
                            def drain1(_i, c):
                                pltpu.make_async_copy(
                                    stage_hbm.at[pl.ds(0, D)],
                                    ring_v.at[pl.ds(0, D)],
                                    semr).wait()
                                return c

                            lax.fori_loop(0, pend, drain1, 0)

                        pend = jnp.where(pend > RING - L, 0, pend)
                        vv = lst2[pl.ds(p0 + j * L, L)]
                        act = (lanes + j * L) < cnt
                        act32 = act.astype(jnp.int32)
                        for lane in range(L):
                            mbit = act32[lane]
                            slot = pend

                            @pl.when(mbit == 1)
                            def _():
                                q = vv[lane] & 511
                                b = vv[lane] >> 15
                                qs = jnp.full((L,), q, jnp.int32)
                                dbase = jnp.full((L,), boff, jnp.int32)
                                for jj in range(D // L):
                                    dv = dbase + lanes + jj * L
                                    vals = plsc.load_gather(qbuf, [dv, qs])
                                    ring_v[pl.ds(slot * RP + jj * L, L)] = vals
                                pltpu.async_copy(
                                    ring_v.at[pl.ds(slot * RP, D)],
                                    stage_hbm.at[pl.ds(b * D, D)],
                                    semr)

                            pend = pend + mbit
                        return pend

                    return lax.fori_loop(0, (cnt + L - 1) // L, scan_seg,
                                         pending)

                return lax.cond((g < NCHUNK) & (cnt > 0), serve,
                                lambda p: p, pending)

            def group(q8, pending):
                qb = q8 * NR
                for r in range(NR):
                    Q = qb + r
                    wait_q(Q, semq[r], r * D)
                    rn = (r + NR - 1) % NR
                    issue(Q + NR - 1, semq[rn], rn * D)
                    pending = process(Q, r * D, pending)
                return pending

            pending = lax.fori_loop(0, NQ // NR, group, 0)

            def final_drain(_i, carry):
                pltpu.make_async_copy(
                    stage_hbm.at[pl.ds(0, D)], ring_v.at[pl.ds(0, D)],
                    semr).wait()
                return carry

            lax.fori_loop(0, pending, final_drain, 0)
            finish()

        def prime_for(tT_hbm):
            def prime():
                def issue0(Q, sem, boff):
                    g = wid + Q * NW
                    col = g * CW

                    @pl.when(g < NCHUNK)
                    def _():
                        for i in range(D // 8):
                            pltpu.async_copy(
                                tT_hbm.at[pl.ds(i * 8, 8), pl.ds(col, UW)],
                                qbuf.at[pl.ds(boff + i * 8, 8), :], sem)

                for r in range(NR - 1):
                    issue0(r, semq[r], r * D)

            return prime

        one_table(uids_hbm, utT_hbm, ustage_hbm,
                  prime_for(utT_hbm), prime_for(itT_hbm))
        one_table(iids_hbm, itT_hbm, istage_hbm,
                  lambda: None, lambda: None)

    return kern


@functools.cache
def _make_dot_kernel(B):
    mesh = plsc.VectorSubcoreMesh(core_axis_name="c", subcore_axis_name="s")
    BPW = B // NW
    GRP = 256

    @functools.partial(
        pl.kernel,
        mesh=mesh,
        out_type=jax.ShapeDtypeStruct((B,), jnp.float32),
        compiler_params=pltpu.CompilerParams(
            needs_layout_passes=False, use_tc_tiling_on_sc=False),
        scratch_types=[
            pltpu.VMEM((GRP * D,), jnp.float32),
            pltpu.VMEM((GRP * D,), jnp.float32),
            pltpu.VMEM((BPW,), jnp.int32),
            pltpu.VMEM((BPW,), jnp.int32),
            pltpu.VMEM((D * D,), jnp.float32),
            pltpu.VMEM((D * D,), jnp.float32),
            pltpu.VMEM((BPW,), jnp.float32),
        ],
    )
    def kern(uids_hbm, iids_hbm, ustage_hbm, istage_hbm, utail_hbm, itail_hbm,
             out_hbm, uv, iv, uidv, iidv, utl, itl, out_v):
        wid = lax.axis_index("s") * NC + lax.axis_index("c")
        base = wid * BPW
        lanes = lax.iota(jnp.int32, L)

        pltpu.sync_copy(uids_hbm.at[pl.ds(base, BPW)], uidv)
        pltpu.sync_copy(iids_hbm.at[pl.ds(base, BPW)], iidv)
        pltpu.sync_copy(utail_hbm, utl)
        pltpu.sync_copy(itail_hbm, itl)

        for grp in range(BPW // GRP):
            g0 = base + grp * GRP
            pltpu.sync_copy(ustage_hbm.at[pl.ds(g0 * D, GRP * D)], uv)
            pltpu.sync_copy(istage_hbm.at[pl.ds(g0 * D, GRP * D)], iv)

            def body(it, carry):
                r0 = it * L
                res = jnp.zeros((L,), jnp.float32)
                for kk in range(L):
                    r = r0 + kk
                    acc = jnp.zeros((L,), jnp.float32)
                    for j in range(D // L):
                        pu = uv[pl.ds(r * D + j * L, L)]
                        pi = iv[pl.ds(r * D + j * L, L)]
                        acc = acc + pu * pi
                    res = jnp.where(lanes == kk, jnp.sum(acc), res)

                ob = grp * GRP + r0
                uidvec = uidv[pl.ds(ob, L)]
                iidvec = iidv[pl.ds(ob, L)]
                tl = (uidvec >= TAIL) | (iidvec >= TAIL)
                tl32 = tl.astype(jnp.int32)
                nt = plsc.all_reduce_population_count(tl)[0]

                @pl.when(nt > 0)
                def _():
                    fixed = res
                    for lane in range(L):
                        @pl.when(tl32[lane] == 1)
                        def _():
                            r = r0 + lane
                            uid = uidvec[lane]
                            iid = iidvec[lane]
                            uo = lax.max(uid - TAIL, 0) * D
                            io = lax.max(iid - TAIL, 0) * D
                            acc2 = jnp.zeros((L,), jnp.float32)
                            for j in range(D // L):
                                us = uv[pl.ds(r * D + j * L, L)]
                                ut = utl[pl.ds(uo + j * L, L)]
                                uu = jnp.where(uid >= TAIL, ut, us)
                                ss = iv[pl.ds(r * D + j * L, L)]
                                st = itl[pl.ds(io + j * L, L)]
                                ii = jnp.where(iid >= TAIL, st, ss)
                                acc2 = acc2 + uu * ii
                            s = jnp.sum(acc2)
                            cur = out_v[pl.ds(ob, L)]
                            out_v[pl.ds(ob, L)] = jnp.where(
                                lanes == lane, s, cur)

                    out_v[pl.ds(ob, L)] = jnp.where(
                        tl, out_v[pl.ds(ob, L)], fixed)

                @pl.when(nt == 0)
                def _():
                    out_v[pl.ds(ob, L)] = res

                return carry

            lax.fori_loop(0, GRP // L, body, 0)

        pltpu.sync_copy(out_v, out_hbm.at[pl.ds(base, BPW)])

    return kern


def kernel(user_ids, item_ids, user_table, item_table):
    B = user_ids.shape[0]
    V = user_table.shape[0]
    uids = user_ids.astype(jnp.int32)
    iids = item_ids.astype(jnp.int32)
    utail = user_table[TAIL:, :].reshape(-1)
    itail = item_table[TAIL:, :].reshape(-1)
    gather = _make_gather_kernel(B, V)
    ustage, istage = gather(uids, iids, user_table.T, item_table.T)
    dot = _make_dot_kernel(B)
    return dot(uids, iids, ustage, istage, utail, itail)


# speedup vs baseline: 1.0263x; 1.0081x over previous
import functools

import jax
import jax.numpy as jnp
from jax import lax
from jax.experimental import pallas as pl
from jax.experimental.pallas import tpu as pltpu
from jax.experimental.pallas import tpu_sc as plsc

D = 64
L = 16
NC = 2
NS = 16
NW = NC * NS
CW = 512
NCHUNK = 999936 // CW
TAIL = 999936
RING = 64
RP = 128


NQ = 62
NR = 2
UW = 512


@functools.cache
def _make_gather_kernel(B, V):
    mesh = plsc.VectorSubcoreMesh(core_axis_name="c", subcore_axis_name="s")
    NV = B // L

    @functools.partial(
        pl.kernel,
        mesh=mesh,
        out_type=(jax.ShapeDtypeStruct((B * D,), jnp.float32),
                  jax.ShapeDtypeStruct((B * D,), jnp.float32)),
        compiler_params=pltpu.CompilerParams(
            needs_layout_passes=False, use_tc_tiling_on_sc=True),
        scratch_types=[
            pltpu.VMEM((B,), jnp.int32),
            pltpu.VMEM((B + L,), jnp.int32),
            pltpu.VMEM((B + L,), jnp.int32),
            pltpu.VMEM((NR * D, UW), jnp.float32),
            pltpu.VMEM((RING * RP,), jnp.float32),
            pltpu.SMEM((64,), jnp.int32),
            pltpu.SMEM((64,), jnp.int32),
        ] + [pltpu.SemaphoreType.DMA] * (NR + 1),
    )
    def kern(uids_hbm, iids_hbm, utT_hbm, itT_hbm, ustage_hbm, istage_hbm,
             ids_v, lst, lst2, qbuf, ring_v, counts_s, pos_s, *sems):
        semq = sems[:NR]
        semr = sems[NR]
        wid = lax.axis_index("s") * NC + lax.axis_index("c")
        lanes = lax.iota(jnp.int32, L)

        def one_table(ids_hbm, tT_hbm, stage_hbm, prime, finish):
            prime()
            pltpu.sync_copy(ids_hbm, ids_v)

            def scan_ids(j, off):
                idv = ids_v[pl.ds(j * L, L)]
                bv = jnp.full((L,), j * L, jnp.int32) + lanes
                keep = ((idv >> 9) & (NW - 1)) == wid
                cnt = plsc.all_reduce_population_count(keep)[0]
                packed = (bv << 15) | ((idv >> 14) << 9) | (idv & 0x1FF)
                plsc.store_compressed(lst.at[pl.ds(off, L)], packed, mask=keep)
                return off + cnt

            n = lax.fori_loop(0, NV, scan_ids, 0)
            nv = (n + L - 1) // L

            def zero(i, c):
                counts_s[i] = 0
                return c

            lax.fori_loop(0, 64, zero, 0)

            def hist(j, c):
                vv = lst[pl.ds(j * L, L)]
                for lane in range(L):
                    kk = (vv[lane] >> 9) & 0x3F
                    inc = jnp.where(j * L + lane < n, 1, 0)
                    counts_s[kk] = counts_s[kk] + inc
                return c

            lax.fori_loop(0, nv, hist, 0)

            def pref(k, run):
                pos_s[k] = run
                return run + counts_s[k]

            lax.fori_loop(0, 64, pref, 0)

            def scat(j, c):
                vv = lst[pl.ds(j * L, L)]
                for lane in range(L):
                    kk = (vv[lane] >> 9) & 0x3F
                    p = pos_s[kk]
                    ok = j * L + lane < n
                    onehot = (lanes == lane) & ok
                    plsc.store_scatter(
                        lst2, [jnp.full((L,), p, jnp.int32)], vv, mask=onehot)
                    pos_s[kk] = p + jnp.where(ok, 1, 0)
                return c

            lax.fori_loop(0, nv, scat, 0)

            def issue(Q, sem, boff):
                g = wid + Q * NW
                col = g * CW

                @pl.when(g < NCHUNK)
                def _():
                    for i in range(D // 8):
                        pltpu.async_copy(
                            tT_hbm.at[pl.ds(i * 8, 8), pl.ds(col, UW)],
                            qbuf.at[pl.ds(boff + i * 8, 8), :], sem)

            def wait_q(Q, sem, boff):
                g = wid + Q * NW
                col = g * CW

                @pl.when(g < NCHUNK)
                def _():
                    for i in range(D // 8):
                        pltpu.make_async_copy(
                            tT_hbm.at[pl.ds(i * 8, 8), pl.ds(col, UW)],
                            qbuf.at[pl.ds(boff + i * 8, 8), :], sem).wait()

            def process(Q, boff, pending):
                g = wid + Q * NW
                cnt = counts_s[Q]
                p0 = pos_s[Q] - cnt

                def serve(pending):
                    def scan_seg(j, pend):
                        @pl.when(pend > RING - L)
                        def _():
                            def drain1(_i, c):
                                pltpu.make_async_copy(
                                    stage_hbm.at[pl.ds(0, D)],
                                    ring_v.at[pl.ds(0, D)],
                                    semr).wait()
                                return c

                            lax.fori_loop(0, pend, drain1, 0)

                        pend = jnp.where(pend > RING - L, 0, pend)
                        vv = lst2[pl.ds(p0 + j * L, L)]
                        act = (lanes + j * L) < cnt
                        act32 = act.astype(jnp.int32)
                        for lane in range(L):
                            mbit = act32[lane]
                            slot = pend

                            @pl.when(mbit == 1)
                            def _():
                                q = vv[lane] & 511
                                b = vv[lane] >> 15
                                qs = jnp.full((L,), q, jnp.int32)
                                dbase = jnp.full((L,), boff, jnp.int32)
                                for jj in range(D // L):
                                    dv = dbase + lanes + jj * L
                                    vals = plsc.load_gather(qbuf, [dv, qs])
                                    ring_v[pl.ds(slot * RP + jj * L, L)] = vals
                                pltpu.async_copy(
                                    ring_v.at[pl.ds(slot * RP, D)],
                                    stage_hbm.at[pl.ds(b * D, D)],
                                    semr)

                            pend = pend + mbit
                        return pend

                    return lax.fori_loop(0, (cnt + L - 1) // L, scan_seg,
                                         pending)

                return lax.cond((g < NCHUNK) & (cnt > 0), serve,
                                lambda p: p, pending)

            def group(q8, pending):
                qb = q8 * NR
                for r in range(NR):
                    Q = qb + r
                    wait_q(Q, semq[r], r * D)
                    rn = (r + NR - 1) % NR
                    issue(Q + NR - 1, semq[rn], rn * D)
                    pending = process(Q, r * D, pending)
                return pending

            pending = lax.fori_loop(0, NQ // NR, group, 0)

            def final_drain(_i, carry):
                pltpu.make_async_copy(
                    stage_hbm.at[pl.ds(0, D)], ring_v.at[pl.ds(0, D)],
                    semr).wait()
                return carry

            lax.fori_loop(0, pending, final_drain, 0)
            finish()

        def prime_for(tT_hbm):
            def prime():
                def issue0(Q, sem, boff):
                    g = wid + Q * NW
                    col = g * CW

                    @pl.when(g < NCHUNK)
                    def _():
                        for i in range(D // 8):
                            pltpu.async_copy(
                                tT_hbm.at[pl.ds(i * 8, 8), pl.ds(col, UW)],
                                qbuf.at[pl.ds(boff + i * 8, 8), :], sem)

                for r in range(NR - 1):
                    issue0(r, semq[r], r * D)

            return prime

        one_table(uids_hbm, utT_hbm, ustage_hbm,
                  prime_for(utT_hbm), prime_for(itT_hbm))
        one_table(iids_hbm, itT_hbm, istage_hbm,
                  lambda: None, lambda: None)

    return kern


@functools.cache
def _make_dot_kernel(B):
    mesh = plsc.VectorSubcoreMesh(core_axis_name="c", subcore_axis_name="s")
    BPW = B // NW
    GRP = 256

    @functools.partial(
        pl.kernel,
        mesh=mesh,
        out_type=jax.ShapeDtypeStruct((B,), jnp.float32),
        compiler_params=pltpu.CompilerParams(
            needs_layout_passes=False, use_tc_tiling_on_sc=False),
        scratch_types=[
            pltpu.VMEM((2 * GRP * D,), jnp.float32),
            pltpu.VMEM((2 * GRP * D,), jnp.float32),
            pltpu.VMEM((BPW,), jnp.int32),
            pltpu.VMEM((BPW,), jnp.int32),
            pltpu.VMEM((D * D,), jnp.float32),
            pltpu.VMEM((D * D,), jnp.float32),
            pltpu.VMEM((BPW,), jnp.float32),
            pltpu.SemaphoreType.DMA,
            pltpu.SemaphoreType.DMA,
        ],
    )
    def kern(uids_hbm, iids_hbm, ustage_hbm, istage_hbm, utail_hbm, itail_hbm,
             out_hbm, uv, iv, uidv, iidv, utl, itl, out_v, semg0, semg1):
        wid = lax.axis_index("s") * NC + lax.axis_index("c")
        base = wid * BPW
        lanes = lax.iota(jnp.int32, L)

        pltpu.sync_copy(uids_hbm.at[pl.ds(base, BPW)], uidv)
        pltpu.sync_copy(iids_hbm.at[pl.ds(base, BPW)], iidv)
        pltpu.sync_copy(utail_hbm, utl)
        pltpu.sync_copy(itail_hbm, itl)

        ngrp = BPW // GRP
        semg = (semg0, semg1)

        def stage_grp(grp, start):
            g0 = base + grp * GRP
            boff = (grp % 2) * GRP * D
            cu = pltpu.make_async_copy(
                ustage_hbm.at[pl.ds(g0 * D, GRP * D)],
                uv.at[pl.ds(boff, GRP * D)], semg[grp % 2])
            ci = pltpu.make_async_copy(
                istage_hbm.at[pl.ds(g0 * D, GRP * D)],
                iv.at[pl.ds(boff, GRP * D)], semg[grp % 2])
            if start:
                cu.start()
                ci.start()
            else:
                cu.wait()
                ci.wait()

        stage_grp(0, True)
        for grp in range(BPW // GRP):
            boff = (grp % 2) * GRP * D
            stage_grp(grp, False)
            if grp + 1 < ngrp:
                stage_grp(grp + 1, True)

            def body(it, carry):
                r0 = it * L
                res = jnp.zeros((L,), jnp.float32)
                for kk in range(L):
                    r = r0 + kk
                    acc = jnp.zeros((L,), jnp.float32)
                    for j in range(D // L):
                        pu = uv[pl.ds(boff + r * D + j * L, L)]
                        pi = iv[pl.ds(boff + r * D + j * L, L)]
                        acc = acc + pu * pi
                    res = jnp.where(lanes == kk, jnp.sum(acc), res)

                ob = grp * GRP + r0
                uidvec = uidv[pl.ds(ob, L)]
                iidvec = iidv[pl.ds(ob, L)]
                tl = (uidvec >= TAIL) | (iidvec >= TAIL)
                tl32 = tl.astype(jnp.int32)
                nt = plsc.all_reduce_population_count(tl)[0]

                @pl.when(nt > 0)
                def _():
                    fixed = res
                    for lane in range(L):
                        @pl.when(tl32[lane] == 1)
                        def _():
                            r = r0 + lane
                            uid = uidvec[lane]
                            iid = iidvec[lane]
                            uo = lax.max(uid - TAIL, 0) * D
                            io = lax.max(iid - TAIL, 0) * D
                            acc2 = jnp.zeros((L,), jnp.float32)
                            for j in range(D // L):
                                us = uv[pl.ds(boff + r * D + j * L, L)]
                                ut = utl[pl.ds(uo + j * L, L)]
                                uu = jnp.where(uid >= TAIL, ut, us)
                                ss = iv[pl.ds(boff + r * D + j * L, L)]
                                st = itl[pl.ds(io + j * L, L)]
                                ii = jnp.where(iid >= TAIL, st, ss)
                                acc2 = acc2 + uu * ii
                            s = jnp.sum(acc2)
                            cur = out_v[pl.ds(ob, L)]
                            out_v[pl.ds(ob, L)] = jnp.where(
                                lanes == lane, s, cur)

                    out_v[pl.ds(ob, L)] = jnp.where(
                        tl, out_v[pl.ds(ob, L)], fixed)

                @pl.when(nt == 0)
                def _():
                    out_v[pl.ds(ob, L)] = res

                return carry

            lax.fori_loop(0, GRP // L, body, 0)

        pltpu.sync_copy(out_v, out_hbm.at[pl.ds(base, BPW)])

    return kern


def kernel(user_ids, item_ids, user_table, item_table):
    B = user_ids.shape[0]
    V = user_table.shape[0]
    uids = user_ids.astype(jnp.int32)
    iids = item_ids.astype(jnp.int32)
    utail = user_table[TAIL:, :].reshape(-1)
    itail = item_table[TAIL:, :].reshape(-1)
    gather = _make_gather_kernel(B, V)
    ustage, istage = gather(uids, iids, user_table.T, item_table.T)
    dot = _make_dot_kernel(B)
    return dot(uids, iids, ustage, istage, utail, itail)
